# Initial kernel scaffold; baseline (speedup 1.0000x reference)
import functools

import jax
import jax.numpy as jnp
from jax import lax
from jax.experimental import pallas as pl
from jax.experimental.pallas import tpu as pltpu
from jax.experimental.pallas import tpu_sc as plsc

B_ROWS, SEQ = 4096, 192
D = 48
NUM_CORES, NUM_SUBCORES = 2, 16
NW = NUM_CORES * NUM_SUBCORES
CHUNK = 128
B = B_ROWS * SEQ
CHUNKS_PER_W = B // (NW * CHUNK)

_mesh = plsc.VectorSubcoreMesh(core_axis_name="c", subcore_axis_name="s")


@functools.partial(
    pl.kernel,
    out_type=jax.ShapeDtypeStruct((NW, CHUNKS_PER_W, CHUNK, D), jnp.float32),
    mesh=_mesh,
    scratch_types=[
        pltpu.VMEM((CHUNKS_PER_W, CHUNK), jnp.int32),
        pltpu.VMEM((CHUNK, D), jnp.float32),
        pltpu.SemaphoreType.DMA,
    ],
)
def _gather_kernel(table_hbm, idx_hbm, out_hbm, idx_v, buf, sem):
    wid = lax.axis_index("s") * NUM_CORES + lax.axis_index("c")
    pltpu.sync_copy(idx_hbm.at[wid], idx_v)

    @pl.loop(0, CHUNKS_PER_W)
    def _(j):
        pltpu.async_copy(table_hbm.at[idx_v.at[j]], buf, sem).wait()
        pltpu.sync_copy(buf, out_hbm.at[wid, j])


def kernel(camera_indices, table):
    idx = camera_indices.reshape(NW, CHUNKS_PER_W, CHUNK).astype(jnp.int32)
    out = _gather_kernel(table, idx)
    return out.reshape(B_ROWS, SEQ, D)


# speedup vs baseline: 3.5386x; 3.5386x over previous
import functools

import jax
import jax.numpy as jnp
from jax import lax
from jax.experimental import pallas as pl
from jax.experimental.pallas import tpu as pltpu
from jax.experimental.pallas import tpu_sc as plsc

B_ROWS, SEQ = 4096, 192
D = 48
NUM_CORES, NUM_SUBCORES = 2, 16
NW = NUM_CORES * NUM_SUBCORES
CHUNK = 128
B = B_ROWS * SEQ
CHUNKS_PER_W = B // (NW * CHUNK)

_mesh = plsc.VectorSubcoreMesh(core_axis_name="c", subcore_axis_name="s")


@functools.partial(
    pl.kernel,
    out_type=jax.ShapeDtypeStruct((NW, CHUNKS_PER_W, CHUNK, D), jnp.float32),
    mesh=_mesh,
    scratch_types=[
        pltpu.VMEM((CHUNKS_PER_W, CHUNK), jnp.int32),
        pltpu.VMEM((CHUNK, D), jnp.float32),
        pltpu.SemaphoreType.DMA,
    ],
    compiler_params=pltpu.CompilerParams(use_tc_tiling_on_sc=False),
)
def _gather_kernel(table_hbm, idx_hbm, out_hbm, idx_v, buf, sem):
    wid = lax.axis_index("s") * NUM_CORES + lax.axis_index("c")
    pltpu.sync_copy(idx_hbm.at[wid], idx_v)

    @pl.loop(0, CHUNKS_PER_W)
    def _(j):
        pltpu.async_copy(table_hbm.at[idx_v.at[j]], buf, sem).wait()
        pltpu.sync_copy(buf, out_hbm.at[wid, j])


def kernel(camera_indices, table):
    idx = camera_indices.reshape(NW, CHUNKS_PER_W, CHUNK).astype(jnp.int32)
    out = _gather_kernel(table, idx)
    return out.reshape(B_ROWS, SEQ, D)


# speedup vs baseline: 3.9558x; 1.1179x over previous
import functools

import jax
import jax.numpy as jnp
from jax import lax
from jax.experimental import pallas as pl
from jax.experimental.pallas import tpu as pltpu
from jax.experimental.pallas import tpu_sc as plsc

B_ROWS, SEQ = 4096, 192
D = 48
NUM_CORES, NUM_SUBCORES = 2, 16
NW = NUM_CORES * NUM_SUBCORES
CHUNK = 128
B = B_ROWS * SEQ
CHUNKS_PER_W = B // (NW * CHUNK)

NBUF = 4
GA = 2

_mesh = plsc.VectorSubcoreMesh(core_axis_name="c", subcore_axis_name="s")


@functools.partial(
    pl.kernel,
    out_type=jax.ShapeDtypeStruct((NW, CHUNKS_PER_W, CHUNK, D), jnp.float32),
    mesh=_mesh,
    scratch_types=[
        pltpu.VMEM((CHUNKS_PER_W, CHUNK), jnp.int32),
        pltpu.VMEM((NBUF, CHUNK, D), jnp.float32),
        pltpu.SemaphoreType.DMA,
        pltpu.SemaphoreType.DMA,
    ],
    compiler_params=pltpu.CompilerParams(use_tc_tiling_on_sc=False),
)
def _gather_kernel(table_hbm, idx_hbm, out_hbm, idx_v, bufs, gsem, wsem):
    wid = lax.axis_index("s") * NUM_CORES + lax.axis_index("c")
    pltpu.sync_copy(idx_hbm.at[wid], idx_v)

    for b in range(GA):
        pltpu.async_copy(table_hbm.at[idx_v.at[b]], bufs.at[b], gsem)

    @pl.loop(0, CHUNKS_PER_W, step=NBUF)
    def _(j0):
        for b in range(NBUF):
            j = j0 + b
            pltpu.make_async_copy(
                table_hbm.at[pl.ds(0, CHUNK)], bufs.at[b], gsem
            ).wait()
            pltpu.async_copy(bufs.at[b], out_hbm.at[wid, j], wsem)
            @pl.when(j >= NBUF - GA)
            def _():
                pltpu.make_async_copy(
                    bufs.at[b], out_hbm.at[wid, 0], wsem
                ).wait()

            @pl.when(j + GA < CHUNKS_PER_W)
            def _():
                pltpu.async_copy(
                    table_hbm.at[idx_v.at[j + GA]],
                    bufs.at[(b + GA) % NBUF],
                    gsem,
                )

    for _ in range(NBUF - GA):
        pltpu.make_async_copy(bufs.at[0], out_hbm.at[wid, 0], wsem).wait()


def kernel(camera_indices, table):
    idx = camera_indices.reshape(NW, CHUNKS_PER_W, CHUNK).astype(jnp.int32)
    out = _gather_kernel(table, idx)
    return out.reshape(B_ROWS, SEQ, D)


# speedup vs baseline: 4.3210x; 1.0923x over previous
import functools

import jax
import jax.numpy as jnp
from jax import lax
from jax.experimental import pallas as pl
from jax.experimental.pallas import tpu as pltpu
from jax.experimental.pallas import tpu_sc as plsc

B_ROWS, SEQ = 4096, 192
D = 48
NUM_CORES, NUM_SUBCORES = 2, 16
NW = NUM_CORES * NUM_SUBCORES
CHUNK = 128
B = B_ROWS * SEQ
CHUNKS_PER_W = B // (NW * CHUNK)

NBUF = 8
GA = 4

_mesh = plsc.VectorSubcoreMesh(core_axis_name="c", subcore_axis_name="s")


@functools.partial(
    pl.kernel,
    out_type=jax.ShapeDtypeStruct((NW, CHUNKS_PER_W, CHUNK, D), jnp.float32),
    mesh=_mesh,
    scratch_types=[
        pltpu.VMEM((CHUNKS_PER_W, CHUNK), jnp.int32),
        pltpu.VMEM((NBUF, CHUNK, D), jnp.float32),
        pltpu.SemaphoreType.DMA,
        pltpu.SemaphoreType.DMA,
    ],
    compiler_params=pltpu.CompilerParams(use_tc_tiling_on_sc=False),
)
def _gather_kernel(table_hbm, idx_hbm, out_hbm, idx_v, bufs, gsem, wsem):
    wid = lax.axis_index("s") * NUM_CORES + lax.axis_index("c")
    pltpu.sync_copy(idx_hbm.at[wid], idx_v)

    for b in range(GA):
        pltpu.async_copy(table_hbm.at[idx_v.at[b]], bufs.at[b], gsem)

    @pl.loop(0, CHUNKS_PER_W, step=NBUF)
    def _(j0):
        for b in range(NBUF):
            j = j0 + b
            pltpu.make_async_copy(
                table_hbm.at[pl.ds(0, CHUNK)], bufs.at[b], gsem
            ).wait()
            pltpu.async_copy(bufs.at[b], out_hbm.at[wid, j], wsem)
            @pl.when(j >= NBUF - GA)
            def _():
                pltpu.make_async_copy(
                    bufs.at[b], out_hbm.at[wid, 0], wsem
                ).wait()

            @pl.when(j + GA < CHUNKS_PER_W)
            def _():
                pltpu.async_copy(
                    table_hbm.at[idx_v.at[j + GA]],
                    bufs.at[(b + GA) % NBUF],
                    gsem,
                )

    for _ in range(NBUF - GA):
        pltpu.make_async_copy(bufs.at[0], out_hbm.at[wid, 0], wsem).wait()


def kernel(camera_indices, table):
    idx = camera_indices.reshape(NW, CHUNKS_PER_W, CHUNK).astype(jnp.int32)
    out = _gather_kernel(table, idx)
    return out.reshape(B_ROWS, SEQ, D)


# speedup vs baseline: 4.3383x; 1.0040x over previous
import functools

import jax
import jax.numpy as jnp
from jax import lax
from jax.experimental import pallas as pl
from jax.experimental.pallas import tpu as pltpu
from jax.experimental.pallas import tpu_sc as plsc

B_ROWS, SEQ = 4096, 192
D = 48
NUM_CORES, NUM_SUBCORES = 2, 16
NW = NUM_CORES * NUM_SUBCORES
ROWS_PER_W = B_ROWS // NW
HALF = SEQ // 2

NBUF = 8
GA = 4

_mesh = plsc.VectorSubcoreMesh(core_axis_name="c", subcore_axis_name="s")


@functools.partial(
    pl.kernel,
    out_type=jax.ShapeDtypeStruct((B_ROWS, SEQ, D), jnp.float32),
    mesh=_mesh,
    scratch_types=[
        pltpu.VMEM((ROWS_PER_W, SEQ), jnp.int32),
        pltpu.VMEM((NBUF, SEQ, D), jnp.float32),
        pltpu.SemaphoreType.DMA,
        pltpu.SemaphoreType.DMA,
    ],
    compiler_params=pltpu.CompilerParams(use_tc_tiling_on_sc=False),
)
def _gather_kernel(table_hbm, idx_hbm, out_hbm, idx_v, bufs, gsem, wsem):
    wid = lax.axis_index("s") * NUM_CORES + lax.axis_index("c")
    base = wid * ROWS_PER_W
    pltpu.sync_copy(idx_hbm.at[pl.ds(base, ROWS_PER_W)], idx_v)

    def start_row_gathers(r, b):
        for h in range(2):
            pltpu.async_copy(
                table_hbm.at[idx_v.at[r, pl.ds(h * HALF, HALF)]],
                bufs.at[b, pl.ds(h * HALF, HALF)],
                gsem,
            )

    for b in range(GA):
        start_row_gathers(b, b)

    @pl.loop(0, ROWS_PER_W, step=NBUF)
    def _(r0):
        for b in range(NBUF):
            r = r0 + b
            pltpu.make_async_copy(
                table_hbm.at[pl.ds(0, SEQ)], bufs.at[b], gsem
            ).wait()
            pltpu.async_copy(bufs.at[b], out_hbm.at[base + r], wsem)

            @pl.when(r >= NBUF - GA)
            def _():
                pltpu.make_async_copy(
                    bufs.at[b], out_hbm.at[base], wsem
                ).wait()

            @pl.when(r + GA < ROWS_PER_W)
            def _():
                start_row_gathers(r + GA, (b + GA) % NBUF)

    for _ in range(NBUF - GA):
        pltpu.make_async_copy(bufs.at[0], out_hbm.at[base], wsem).wait()


def kernel(camera_indices, table):
    return _gather_kernel(table, camera_indices.astype(jnp.int32))
